# baseline (speedup 1.0000x reference)
import functools

import jax
import jax.numpy as jnp
from jax import lax
from jax.experimental import pallas as pl
from jax.experimental.pallas import tpu as pltpu
from jax.experimental.pallas import tpu_sc as plsc

N = 10000
E = 320000
D_IN = 128
D_EDGE = 16
LATENT = 64

NC = 2
NS = 16
NW = NC * NS
CB = 80
NBUF = 3
G0 = 189
G1 = 63
E_PAD = NS * (G0 + G1) * CB
EH = E_PAD // 2
HC = CB // 2

_PERM_LO = [c * 32 + k for c in range(D_IN // 32) for k in range(16)]
_PERM_HI = [c * 32 + 16 + k for c in range(D_IN // 32) for k in range(16)]
N_PAD = 10112
ROWS_PER_TILE = N_PAD // NS
WB_CHUNKS = [(j * CB, CB) for j in range(ROWS_PER_TILE // CB)]
if ROWS_PER_TILE % CB:
    WB_CHUNKS.append((ROWS_PER_TILE - ROWS_PER_TILE % CB, ROWS_PER_TILE % CB))


def _pack_bf16_pair(a, b):
    au = lax.bitcast_convert_type(a, jnp.uint32)
    bu = lax.bitcast_convert_type(b, jnp.uint32)
    ar = au + jnp.uint32(0x7FFF) + ((au >> 16) & jnp.uint32(1))
    br = bu + jnp.uint32(0x7FFF) + ((bu >> 16) & jnp.uint32(1))
    packed = (ar >> 16) | (br & jnp.uint32(0xFFFF0000))
    return lax.bitcast_convert_type(packed, jnp.int32)


def _edge_embed_body(eaA_ref, eaB_ref, w_ref, b_ref, e_ref):
    h = D_IN // 2
    fullA = (jnp.dot(eaA_ref[...], w_ref[...], preferred_element_type=jnp.float32)
             + b_ref[...])
    fullB = (jnp.dot(eaB_ref[...], w_ref[...], preferred_element_type=jnp.float32)
             + b_ref[...])
    e_ref[...] = jnp.concatenate(
        [_pack_bf16_pair(fullA[:, :h], fullA[:, h:]),
         _pack_bf16_pair(fullB[:, :h], fullB[:, h:])], axis=1)


def _edge_embed(ea_pad, wt, b):
    BE = 2016
    nblk = EH // BE
    grid = (nblk,)
    full = lambda shape: pl.BlockSpec(shape, lambda i: (0, 0))
    return pl.pallas_call(
        _edge_embed_body,
        grid=grid,
        in_specs=[
            pl.BlockSpec((BE, D_EDGE), lambda i: (i, 0)),
            pl.BlockSpec((BE, D_EDGE), lambda i, n=nblk: (n + i, 0)),
            full((D_EDGE, D_IN)), full((1, D_IN)),
        ],
        out_specs=pl.BlockSpec((BE, D_IN), lambda i: (i, 0)),
        out_shape=jax.ShapeDtypeStruct((EH, D_IN), jnp.int32),
    )(ea_pad, ea_pad, wt, b)


def _zero_buf(buf, rows, d):
    z = jnp.zeros((16,), jnp.float32)

    def row(r, _):
        for c in range(d // 16):
            buf[r, pl.ds(c * 16, 16)] = z
        return 0

    lax.fori_loop(0, rows, row, 0, unroll=False)


def _relu_add_packed_e(dst, eb):
    def row(q, _):
        for half in range(2):
            p_base = half * HC
            w_base = half * (D_IN // 2)
            for c in range(D_IN // 32):
                w = eb[q, pl.ds(w_base + c * 16, 16)]
                lo = lax.bitcast_convert_type(w << 16, jnp.float32)
                hi = lax.bitcast_convert_type(
                    w & jnp.int32(-65536), jnp.float32)
                sl0 = pl.ds(c * 32, 16)
                sl1 = pl.ds(c * 32 + 16, 16)
                dst[p_base + q, sl0] = jnp.maximum(dst[p_base + q, sl0] + lo, 0.0)
                dst[p_base + q, sl1] = jnp.maximum(dst[p_base + q, sl1] + hi, 0.0)
        return 0

    lax.fori_loop(0, HC, row, 0, unroll=False)


def _sc_aggregate(table, src, dst, e):
    mesh = plsc.VectorSubcoreMesh(core_axis_name="c", subcore_axis_name="s",
                                  num_cores=NC, num_subcores=NS)

    @functools.partial(
        pl.kernel,
        out_type=jax.ShapeDtypeStruct((NC, N_PAD, D_IN), jnp.float32),
        mesh=mesh,
        scratch_types=(
            [pltpu.VMEM((CB,), jnp.int32) for _ in range(NBUF)]
            + [pltpu.VMEM((CB,), jnp.int32) for _ in range(NBUF)]
            + [pltpu.VMEM((CB, D_IN), jnp.float32) for _ in range(NBUF)]
            + [pltpu.VMEM((HC, D_IN), jnp.int32) for _ in range(NBUF)]
            + [pltpu.VMEM_SHARED((N_PAD, D_IN), jnp.float32)]
            + [pltpu.SemaphoreType.DMA for _ in range(3 * NBUF)]
        ),
    )
    def k(tab_hbm, src_hbm, dst_hbm, e_hbm, out_hbm, *bufs):
        sidx = bufs[0:NBUF]
        didx = bufs[NBUF:2 * NBUF]
        rows = bufs[2 * NBUF:3 * NBUF]
        ebuf = bufs[3 * NBUF:4 * NBUF]
        acc_sh = bufs[4 * NBUF]
        sem_s = bufs[4 * NBUF + 1:5 * NBUF + 1]
        sem_in = bufs[5 * NBUF + 1:6 * NBUF + 1]
        sem_g = bufs[6 * NBUF + 1:7 * NBUF + 1]

        cid = lax.axis_index("c")
        sid = lax.axis_index("s")
        gc = jnp.where(cid == 0, G0, G1)
        cbase = jnp.where(cid == 0, sid * G0, NS * G0 + sid * G1)
        ebase = cbase * CB
        hbase = cbase * HC

        def issue_sidx(g, b):
            pltpu.async_copy(src_hbm.at[pl.ds(ebase + g * CB, CB)],
                             sidx[b], sem_s[b])

        def issue_de(g, b):
            pltpu.async_copy(dst_hbm.at[pl.ds(ebase + g * CB, CB)],
                             didx[b], sem_in[b])
            pltpu.async_copy(e_hbm.at[pl.ds(hbase + g * HC, HC), :],
                             ebuf[b], sem_in[b])

        def wait_sidx(b):
            pltpu.make_async_copy(src_hbm.at[pl.ds(0, CB)], sidx[b],
                                  sem_s[b]).wait()

        def issue_gather(b):
            pltpu.async_copy(tab_hbm.at[sidx[b]], rows[b], sem_g[b])

        def wait_gather(b):
            pltpu.make_async_copy(tab_hbm.at[sidx[b]], rows[b], sem_g[b]).wait()

        def wait_in(b):
            pltpu.make_async_copy(dst_hbm.at[pl.ds(0, CB)], didx[b],
                                  sem_in[b]).wait()
            pltpu.make_async_copy(e_hbm.at[pl.ds(0, HC), :], ebuf[b],
                                  sem_in[b]).wait()

        _zero_buf(rows[0], CB, D_IN)
        row0 = sid * ROWS_PER_TILE
        for off, nr in WB_CHUNKS:
            pltpu.sync_copy(rows[0].at[pl.ds(0, nr), :],
                            acc_sh.at[pl.ds(row0 + off, nr), :])
        plsc.subcore_barrier()

        for j in range(NBUF):
            issue_sidx(j, j)
            issue_de(j, j)
        for j in range(NBUF - 1):
            wait_sidx(j)
            issue_gather(j)

        def step(t, _):
            for b in range(NBUF):
                g = NBUF * t + b
                bl = (b + NBUF - 1) % NBUF

                @pl.when(g + NBUF - 1 < gc)
                def _():
                    wait_sidx(bl)
                    issue_gather(bl)

                wait_gather(b)

                @pl.when(g + NBUF < gc)
                def _():
                    issue_sidx(g + NBUF, b)

                wait_in(b)
                _relu_add_packed_e(rows[b], ebuf[b])
                pltpu.sync_copy(rows[b], acc_sh.at[didx[b]], add=True)

                @pl.when(g + NBUF < gc)
                def _():
                    issue_de(g + NBUF, b)
            return 0

        lax.fori_loop(0, gc // NBUF, step, 0, unroll=False)
        plsc.subcore_barrier()

        for off, nr in WB_CHUNKS:
            sl = pl.ds(row0 + off, nr)
            st = pl.ds(0, nr)
            pltpu.sync_copy(acc_sh.at[sl, :], rows[0].at[st, :])
            pltpu.sync_copy(rows[0].at[st, :], out_hbm.at[cid, sl, :])

    return k(table, src, dst, e)


def _mlp1_body(x_ref, a0_ref, a1_ref, w1_ref, b1_ref, w2_ref, b2_ref, hh_ref):
    s = x_ref[...] + a0_ref[0] + a1_ref[0]
    t = jnp.maximum(jnp.dot(s, w1_ref[...], preferred_element_type=jnp.float32)
                    + b1_ref[...], 0.0)
    u = jnp.dot(t, w2_ref[...], preferred_element_type=jnp.float32) + b2_ref[...]
    h = jnp.maximum(u, 0.0)
    hh_ref[...] = jnp.concatenate([h, h], axis=1)


def _mlp1(x, acc, w1t, b1, w2t, b2):
    BN = 1000
    grid = (N // BN,)
    full = lambda shape: pl.BlockSpec(shape, lambda i: (0, 0))
    return pl.pallas_call(
        _mlp1_body,
        grid=grid,
        in_specs=[
            pl.BlockSpec((BN, D_IN), lambda i: (i, 0)),
            pl.BlockSpec((1, BN, D_IN), lambda i: (0, i, 0)),
            pl.BlockSpec((1, BN, D_IN), lambda i: (1, i, 0)),
            full((D_IN, LATENT)), full((1, LATENT)),
            full((LATENT, LATENT)), full((1, LATENT)),
        ],
        out_specs=pl.BlockSpec((BN, D_IN), lambda i: (i, 0)),
        out_shape=jax.ShapeDtypeStruct((N, D_IN), jnp.float32),
    )(x, acc, acc, w1t, b1, w2t, b2)


def _mlp23_body(hh_ref, a0_ref, a1_ref,
                mw1_ref, mb1_ref, mw2_ref, mb2_ref,
                lw1_ref, lb1_ref, lw2_ref, lb2_ref, mu_ref, ls_ref):
    h = hh_ref[:, :LATENT]
    a = a0_ref[0] + a1_ref[0]
    s2 = h + a[:, :LATENT]
    t2 = jnp.maximum(jnp.dot(s2, mw1_ref[...], preferred_element_type=jnp.float32)
                     + mb1_ref[...], 0.0)
    mu_ref[...] = jnp.dot(t2, mw2_ref[...], preferred_element_type=jnp.float32) + mb2_ref[...]
    s3 = h + a[:, LATENT:]
    t3 = jnp.maximum(jnp.dot(s3, lw1_ref[...], preferred_element_type=jnp.float32)
                     + lb1_ref[...], 0.0)
    u3 = jnp.dot(t3, lw2_ref[...], preferred_element_type=jnp.float32) + lb2_ref[...]
    ls_ref[...] = jnp.clip(u3, -10.0, 10.0)


def _mlp23(hh, acc, mw1t, mb1, mw2t, mb2, lw1t, lb1, lw2t, lb2):
    BN = 1000
    grid = (N // BN,)
    full = lambda shape: pl.BlockSpec(shape, lambda i: (0, 0))
    return pl.pallas_call(
        _mlp23_body,
        grid=grid,
        in_specs=[
            pl.BlockSpec((BN, D_IN), lambda i: (i, 0)),
            pl.BlockSpec((1, BN, D_IN), lambda i: (0, i, 0)),
            pl.BlockSpec((1, BN, D_IN), lambda i: (1, i, 0)),
            full((LATENT, LATENT)), full((1, LATENT)),
            full((LATENT, LATENT)), full((1, LATENT)),
            full((LATENT, LATENT)), full((1, LATENT)),
            full((LATENT, LATENT)), full((1, LATENT)),
        ],
        out_specs=[
            pl.BlockSpec((BN, LATENT), lambda i: (i, 0)),
            pl.BlockSpec((BN, LATENT), lambda i: (i, 0)),
        ],
        out_shape=[
            jax.ShapeDtypeStruct((N, LATENT), jnp.float32),
            jax.ShapeDtypeStruct((N, LATENT), jnp.float32),
        ],
    )(hh, acc, acc, mw1t, mb1, mw2t, mb2, lw1t, lb1, lw2t, lb2)


def _pair_order(v):
    return jnp.stack([v[:EH].reshape(-1, HC), v[EH:].reshape(-1, HC)],
                     axis=1).reshape(-1)


def kernel(x, edge_index, edge_attr,
           le1_W, le1_b, c1_W1, c1_b1, c1_W2, c1_b2,
           le2_W, le2_b, mu_W1, mu_b1, mu_W2, mu_b2,
           le3_W, le3_b, ls_W1, ls_b1, ls_W2, ls_b2):
    src = edge_index[0].astype(jnp.int32)
    dst = edge_index[1].astype(jnp.int32)
    pad = E_PAD - E
    src = jnp.pad(src, (0, pad))
    dst = jnp.concatenate(
        [dst, N + jnp.arange(pad, dtype=jnp.int32) % (N_PAD - N)])
    ea = jnp.pad(edge_attr, ((0, pad), (0, 0)))
    srcp = _pair_order(src)
    dstp = _pair_order(dst)

    plo = jnp.array(_PERM_LO, dtype=jnp.int32)
    phi = jnp.array(_PERM_HI, dtype=jnp.int32)
    w1t = le1_W.T
    w23t = jnp.concatenate([le2_W.T, le3_W.T], axis=1)
    b23 = jnp.concatenate([le2_b, le3_b])
    w1p = jnp.concatenate([w1t[:, plo], w1t[:, phi]], axis=1)
    b1p = jnp.concatenate([le1_b[plo], le1_b[phi]]).reshape(1, -1)
    w23p = jnp.concatenate([w23t[:, plo], w23t[:, phi]], axis=1)
    b23p = jnp.concatenate([b23[plo], b23[phi]]).reshape(1, -1)
    e1 = _edge_embed(ea, w1p, b1p)

    acc1 = _sc_aggregate(x, srcp, dstp, e1)
    e23 = _edge_embed(ea, w23p, b23p)
    hh = _mlp1(x, acc1, c1_W1.T, c1_b1.reshape(1, -1), c1_W2.T, c1_b2.reshape(1, -1))
    acc23 = _sc_aggregate(hh, srcp, dstp, e23)
    mu, logstd = _mlp23(
        hh, acc23,
        mu_W1.T, mu_b1.reshape(1, -1), mu_W2.T, mu_b2.reshape(1, -1),
        ls_W1.T, ls_b1.reshape(1, -1), ls_W2.T, ls_b2.reshape(1, -1),
    )
    return (mu, logstd)


# speedup vs baseline: 1.0489x; 1.0489x over previous
import functools

import jax
import jax.numpy as jnp
from jax import lax
from jax.experimental import pallas as pl
from jax.experimental.pallas import tpu as pltpu
from jax.experimental.pallas import tpu_sc as plsc

N = 10000
E = 320000
D_IN = 128
D_EDGE = 16
LATENT = 64

NC = 2
NS = 16
NW = NC * NS
CB = 80
NBUF = 3
G0 = 189
G1 = 63
E_PAD = NS * (G0 + G1) * CB
EH = E_PAD // 2
HC = CB // 2

_PERM_LO = [c * 32 + k for c in range(D_IN // 32) for k in range(16)]
_PERM_HI = [c * 32 + 16 + k for c in range(D_IN // 32) for k in range(16)]
N_PAD = 10112
ROWS_PER_TILE = N_PAD // NS
WB_CHUNKS = [(j * CB, CB) for j in range(ROWS_PER_TILE // CB)]
if ROWS_PER_TILE % CB:
    WB_CHUNKS.append((ROWS_PER_TILE - ROWS_PER_TILE % CB, ROWS_PER_TILE % CB))


def _pack_bf16_pair(a, b):
    au = lax.bitcast_convert_type(a, jnp.uint32)
    bu = lax.bitcast_convert_type(b, jnp.uint32)
    ar = au + jnp.uint32(0x7FFF) + ((au >> 16) & jnp.uint32(1))
    br = bu + jnp.uint32(0x7FFF) + ((bu >> 16) & jnp.uint32(1))
    packed = (ar >> 16) | (br & jnp.uint32(0xFFFF0000))
    return lax.bitcast_convert_type(packed, jnp.int32)


def _edge_embed_body(eaA_ref, eaB_ref, w_ref, b_ref, e1_ref, e23_ref):
    h = D_IN // 2
    fullA = (jnp.dot(eaA_ref[...], w_ref[...], preferred_element_type=jnp.float32)
             + b_ref[...])
    fullB = (jnp.dot(eaB_ref[...], w_ref[...], preferred_element_type=jnp.float32)
             + b_ref[...])
    e1_ref[...] = jnp.concatenate(
        [_pack_bf16_pair(fullA[:, :h], fullA[:, h:2 * h]),
         _pack_bf16_pair(fullB[:, :h], fullB[:, h:2 * h])], axis=1)
    e23_ref[...] = jnp.concatenate(
        [_pack_bf16_pair(fullA[:, 2 * h:3 * h], fullA[:, 3 * h:]),
         _pack_bf16_pair(fullB[:, 2 * h:3 * h], fullB[:, 3 * h:])], axis=1)


def _edge_embed(ea_pad, wall, ball):
    BE = 2016
    nblk = EH // BE
    grid = (nblk,)
    full = lambda shape: pl.BlockSpec(shape, lambda i: (0, 0))
    return pl.pallas_call(
        _edge_embed_body,
        grid=grid,
        in_specs=[
            pl.BlockSpec((BE, D_EDGE), lambda i: (i, 0)),
            pl.BlockSpec((BE, D_EDGE), lambda i, n=nblk: (n + i, 0)),
            full((D_EDGE, 2 * D_IN)), full((1, 2 * D_IN)),
        ],
        out_specs=[
            pl.BlockSpec((BE, D_IN), lambda i: (i, 0)),
            pl.BlockSpec((BE, D_IN), lambda i: (i, 0)),
        ],
        out_shape=[
            jax.ShapeDtypeStruct((EH, D_IN), jnp.int32),
            jax.ShapeDtypeStruct((EH, D_IN), jnp.int32),
        ],
    )(ea_pad, ea_pad, wall, ball)


def _zero_buf(buf, rows, d):
    z = jnp.zeros((16,), jnp.float32)

    def row(r, _):
        for c in range(d // 16):
            buf[r, pl.ds(c * 16, 16)] = z
        return 0

    lax.fori_loop(0, rows, row, 0, unroll=False)


def _relu_add_packed_e(dst, eb):
    def row(q, _):
        for half in range(2):
            p_base = half * HC
            w_base = half * (D_IN // 2)
            for c in range(D_IN // 32):
                w = eb[q, pl.ds(w_base + c * 16, 16)]
                lo = lax.bitcast_convert_type(w << 16, jnp.float32)
                hi = lax.bitcast_convert_type(
                    w & jnp.int32(-65536), jnp.float32)
                sl0 = pl.ds(c * 32, 16)
                sl1 = pl.ds(c * 32 + 16, 16)
                dst[p_base + q, sl0] = jnp.maximum(dst[p_base + q, sl0] + lo, 0.0)
                dst[p_base + q, sl1] = jnp.maximum(dst[p_base + q, sl1] + hi, 0.0)
        return 0

    lax.fori_loop(0, HC, row, 0, unroll=False)


def _sc_aggregate(table, src, dst, e):
    mesh = plsc.VectorSubcoreMesh(core_axis_name="c", subcore_axis_name="s",
                                  num_cores=NC, num_subcores=NS)

    @functools.partial(
        pl.kernel,
        out_type=jax.ShapeDtypeStruct((NC, N_PAD, D_IN), jnp.float32),
        mesh=mesh,
        scratch_types=(
            [pltpu.VMEM((CB,), jnp.int32) for _ in range(NBUF)]
            + [pltpu.VMEM((CB,), jnp.int32) for _ in range(NBUF)]
            + [pltpu.VMEM((CB, D_IN), jnp.float32) for _ in range(NBUF)]
            + [pltpu.VMEM((HC, D_IN), jnp.int32) for _ in range(NBUF)]
            + [pltpu.VMEM_SHARED((N_PAD, D_IN), jnp.float32)]
            + [pltpu.SemaphoreType.DMA for _ in range(3 * NBUF)]
        ),
    )
    def k(tab_hbm, src_hbm, dst_hbm, e_hbm, out_hbm, *bufs):
        sidx = bufs[0:NBUF]
        didx = bufs[NBUF:2 * NBUF]
        rows = bufs[2 * NBUF:3 * NBUF]
        ebuf = bufs[3 * NBUF:4 * NBUF]
        acc_sh = bufs[4 * NBUF]
        sem_s = bufs[4 * NBUF + 1:5 * NBUF + 1]
        sem_in = bufs[5 * NBUF + 1:6 * NBUF + 1]
        sem_g = bufs[6 * NBUF + 1:7 * NBUF + 1]

        cid = lax.axis_index("c")
        sid = lax.axis_index("s")
        gc = jnp.where(cid == 0, G0, G1)
        cbase = jnp.where(cid == 0, sid * G0, NS * G0 + sid * G1)
        ebase = cbase * CB
        hbase = cbase * HC

        def issue_sidx(g, b):
            pltpu.async_copy(src_hbm.at[pl.ds(ebase + g * CB, CB)],
                             sidx[b], sem_s[b])

        def issue_de(g, b):
            pltpu.async_copy(dst_hbm.at[pl.ds(ebase + g * CB, CB)],
                             didx[b], sem_in[b])
            pltpu.async_copy(e_hbm.at[pl.ds(hbase + g * HC, HC), :],
                             ebuf[b], sem_in[b])

        def wait_sidx(b):
            pltpu.make_async_copy(src_hbm.at[pl.ds(0, CB)], sidx[b],
                                  sem_s[b]).wait()

        def issue_gather(b):
            pltpu.async_copy(tab_hbm.at[sidx[b]], rows[b], sem_g[b])

        def wait_gather(b):
            pltpu.make_async_copy(tab_hbm.at[sidx[b]], rows[b], sem_g[b]).wait()

        def wait_in(b):
            pltpu.make_async_copy(dst_hbm.at[pl.ds(0, CB)], didx[b],
                                  sem_in[b]).wait()
            pltpu.make_async_copy(e_hbm.at[pl.ds(0, HC), :], ebuf[b],
                                  sem_in[b]).wait()

        _zero_buf(rows[0], CB, D_IN)
        row0 = sid * ROWS_PER_TILE
        for off, nr in WB_CHUNKS:
            pltpu.sync_copy(rows[0].at[pl.ds(0, nr), :],
                            acc_sh.at[pl.ds(row0 + off, nr), :])
        plsc.subcore_barrier()

        for j in range(NBUF):
            issue_sidx(j, j)
            issue_de(j, j)
        for j in range(NBUF - 1):
            wait_sidx(j)
            issue_gather(j)

        def step(t, _):
            for b in range(NBUF):
                g = NBUF * t + b
                bl = (b + NBUF - 1) % NBUF

                @pl.when(g + NBUF - 1 < gc)
                def _():
                    wait_sidx(bl)
                    issue_gather(bl)

                wait_gather(b)

                @pl.when(g + NBUF < gc)
                def _():
                    issue_sidx(g + NBUF, b)

                wait_in(b)
                _relu_add_packed_e(rows[b], ebuf[b])
                pltpu.sync_copy(rows[b], acc_sh.at[didx[b]], add=True)

                @pl.when(g + NBUF < gc)
                def _():
                    issue_de(g + NBUF, b)
            return 0

        lax.fori_loop(0, gc // NBUF, step, 0, unroll=False)
        plsc.subcore_barrier()

        for off, nr in WB_CHUNKS:
            sl = pl.ds(row0 + off, nr)
            st = pl.ds(0, nr)
            pltpu.sync_copy(acc_sh.at[sl, :], rows[0].at[st, :])
            pltpu.sync_copy(rows[0].at[st, :], out_hbm.at[cid, sl, :])

    return k(table, src, dst, e)


def _mlp1_body(x_ref, a0_ref, a1_ref, w1_ref, b1_ref, w2_ref, b2_ref, hh_ref):
    s = x_ref[...] + a0_ref[0] + a1_ref[0]
    t = jnp.maximum(jnp.dot(s, w1_ref[...], preferred_element_type=jnp.float32)
                    + b1_ref[...], 0.0)
    u = jnp.dot(t, w2_ref[...], preferred_element_type=jnp.float32) + b2_ref[...]
    h = jnp.maximum(u, 0.0)
    hh_ref[...] = jnp.concatenate([h, h], axis=1)


def _mlp1(x, acc, w1t, b1, w2t, b2):
    BN = 1000
    grid = (N // BN,)
    full = lambda shape: pl.BlockSpec(shape, lambda i: (0, 0))
    return pl.pallas_call(
        _mlp1_body,
        grid=grid,
        in_specs=[
            pl.BlockSpec((BN, D_IN), lambda i: (i, 0)),
            pl.BlockSpec((1, BN, D_IN), lambda i: (0, i, 0)),
            pl.BlockSpec((1, BN, D_IN), lambda i: (1, i, 0)),
            full((D_IN, LATENT)), full((1, LATENT)),
            full((LATENT, LATENT)), full((1, LATENT)),
        ],
        out_specs=pl.BlockSpec((BN, D_IN), lambda i: (i, 0)),
        out_shape=jax.ShapeDtypeStruct((N, D_IN), jnp.float32),
    )(x, acc, acc, w1t, b1, w2t, b2)


def _mlp23_body(hh_ref, a0_ref, a1_ref,
                mw1_ref, mb1_ref, mw2_ref, mb2_ref,
                lw1_ref, lb1_ref, lw2_ref, lb2_ref, mu_ref, ls_ref):
    h = hh_ref[:, :LATENT]
    a = a0_ref[0] + a1_ref[0]
    s2 = h + a[:, :LATENT]
    t2 = jnp.maximum(jnp.dot(s2, mw1_ref[...], preferred_element_type=jnp.float32)
                     + mb1_ref[...], 0.0)
    mu_ref[...] = jnp.dot(t2, mw2_ref[...], preferred_element_type=jnp.float32) + mb2_ref[...]
    s3 = h + a[:, LATENT:]
    t3 = jnp.maximum(jnp.dot(s3, lw1_ref[...], preferred_element_type=jnp.float32)
                     + lb1_ref[...], 0.0)
    u3 = jnp.dot(t3, lw2_ref[...], preferred_element_type=jnp.float32) + lb2_ref[...]
    ls_ref[...] = jnp.clip(u3, -10.0, 10.0)


def _mlp23(hh, acc, mw1t, mb1, mw2t, mb2, lw1t, lb1, lw2t, lb2):
    BN = 1000
    grid = (N // BN,)
    full = lambda shape: pl.BlockSpec(shape, lambda i: (0, 0))
    return pl.pallas_call(
        _mlp23_body,
        grid=grid,
        in_specs=[
            pl.BlockSpec((BN, D_IN), lambda i: (i, 0)),
            pl.BlockSpec((1, BN, D_IN), lambda i: (0, i, 0)),
            pl.BlockSpec((1, BN, D_IN), lambda i: (1, i, 0)),
            full((LATENT, LATENT)), full((1, LATENT)),
            full((LATENT, LATENT)), full((1, LATENT)),
            full((LATENT, LATENT)), full((1, LATENT)),
            full((LATENT, LATENT)), full((1, LATENT)),
        ],
        out_specs=[
            pl.BlockSpec((BN, LATENT), lambda i: (i, 0)),
            pl.BlockSpec((BN, LATENT), lambda i: (i, 0)),
        ],
        out_shape=[
            jax.ShapeDtypeStruct((N, LATENT), jnp.float32),
            jax.ShapeDtypeStruct((N, LATENT), jnp.float32),
        ],
    )(hh, acc, acc, mw1t, mb1, mw2t, mb2, lw1t, lb1, lw2t, lb2)


def _pair_order(v):
    return jnp.stack([v[:EH].reshape(-1, HC), v[EH:].reshape(-1, HC)],
                     axis=1).reshape(-1)


def kernel(x, edge_index, edge_attr,
           le1_W, le1_b, c1_W1, c1_b1, c1_W2, c1_b2,
           le2_W, le2_b, mu_W1, mu_b1, mu_W2, mu_b2,
           le3_W, le3_b, ls_W1, ls_b1, ls_W2, ls_b2):
    src = edge_index[0].astype(jnp.int32)
    dst = edge_index[1].astype(jnp.int32)
    pad = E_PAD - E
    src = jnp.pad(src, (0, pad))
    dst = jnp.concatenate(
        [dst, N + jnp.arange(pad, dtype=jnp.int32) % (N_PAD - N)])
    ea = jnp.pad(edge_attr, ((0, pad), (0, 0)))
    srcp = _pair_order(src)
    dstp = _pair_order(dst)

    plo = jnp.array(_PERM_LO, dtype=jnp.int32)
    phi = jnp.array(_PERM_HI, dtype=jnp.int32)
    w1t = le1_W.T
    w23t = jnp.concatenate([le2_W.T, le3_W.T], axis=1)
    b23 = jnp.concatenate([le2_b, le3_b])
    wall = jnp.concatenate(
        [w1t[:, plo], w1t[:, phi], w23t[:, plo], w23t[:, phi]], axis=1)
    ball = jnp.concatenate(
        [le1_b[plo], le1_b[phi], b23[plo], b23[phi]]).reshape(1, -1)
    e1, e23 = _edge_embed(ea, wall, ball)

    acc1 = _sc_aggregate(x, srcp, dstp, e1)
    hh = _mlp1(x, acc1, c1_W1.T, c1_b1.reshape(1, -1), c1_W2.T, c1_b2.reshape(1, -1))
    acc23 = _sc_aggregate(hh, srcp, dstp, e23)
    mu, logstd = _mlp23(
        hh, acc23,
        mu_W1.T, mu_b1.reshape(1, -1), mu_W2.T, mu_b2.reshape(1, -1),
        ls_W1.T, ls_b1.reshape(1, -1), ls_W2.T, ls_b2.reshape(1, -1),
    )
    return (mu, logstd)
